# baseline (speedup 1.0000x reference)
import functools

import jax
import jax.numpy as jnp
from jax import lax
from jax.experimental import pallas as pl
from jax.experimental.pallas import tpu as pltpu
from jax.experimental.pallas import tpu_sc as plsc

N = 10000
E = 320000
D_IN = 128
D_HID = 128
D_OUT = 40
D_PAD = 64

CH = 125
NROWS = E // CH
ROWS_PER_SUB = NROWS // 16
NP = 10240
NODES_PER_S = NP // 16
ZCH = 128


def _sc_aggregate(dh, h2, srcm, dstm):
    mesh = plsc.VectorSubcoreMesh(core_axis_name="c", subcore_axis_name="s")

    @functools.partial(
        pl.kernel,
        out_type=jax.ShapeDtypeStruct((2, NP, dh), jnp.float32),
        mesh=mesh,
        compiler_params=pltpu.CompilerParams(use_tc_tiling_on_sc=False),
        scratch_types=[
            pltpu.VMEM((ROWS_PER_SUB, CH), jnp.int32),
            pltpu.VMEM((ROWS_PER_SUB, CH), jnp.int32),
            pltpu.VMEM((CH, dh), jnp.float32),
            pltpu.VMEM((CH, dh), jnp.float32),
            pltpu.VMEM((ZCH, dh), jnp.float32),
            pltpu.VMEM_SHARED((NP, dh), jnp.float32),
            pltpu.SemaphoreType.DMA,
            pltpu.SemaphoreType.DMA,
            pltpu.SemaphoreType.DMA,
            pltpu.SemaphoreType.DMA,
        ],
    )
    def agg_kernel(h_hbm, src_hbm, dst_hbm, out_hbm,
                   sidx, didx, rows_a, rows_b, zbuf, acc, gs_a, gs_b, ss_a, ss_b):
        cid = lax.axis_index("c")
        sid = lax.axis_index("s")
        row0 = sid * ROWS_PER_SUB

        pltpu.sync_copy(src_hbm.at[pl.ds(row0, ROWS_PER_SUB)], sidx)
        pltpu.sync_copy(dst_hbm.at[pl.ds(row0, ROWS_PER_SUB)], didx)

        zv = jnp.zeros((16,), jnp.float32)

        @pl.loop(0, ZCH)
        def _(i):
            @pl.loop(0, dh, step=16)
            def _(j):
                zbuf[i, pl.ds(j, 16)] = zv

        @pl.loop(0, NODES_PER_S // ZCH)
        def _(k):
            pltpu.sync_copy(zbuf, acc.at[pl.ds(sid * NODES_PER_S + k * ZCH, ZCH)])

        plsc.subcore_barrier()

        h_slab = h_hbm.at[cid]
        pltpu.async_copy(h_slab.at[sidx.at[0]], rows_a, gs_a)

        def _wait_scatter(buf, sem):
            pltpu.make_async_copy(buf, acc.at[didx.at[0]], sem).wait()

        @pl.loop(0, ROWS_PER_SUB, step=2)
        def _(j):
            pltpu.make_async_copy(h_slab.at[sidx.at[j]], rows_a, gs_a).wait()
            pltpu.async_copy(rows_a, acc.at[didx.at[j]], ss_a, add=True)

            @pl.when(j > 0)
            def _():
                _wait_scatter(rows_b, ss_b)

            pltpu.async_copy(h_slab.at[sidx.at[j + 1]], rows_b, gs_b)
            pltpu.make_async_copy(h_slab.at[sidx.at[j]], rows_b, gs_b).wait()
            pltpu.async_copy(rows_b, acc.at[didx.at[j + 1]], ss_b, add=True)
            _wait_scatter(rows_a, ss_a)

            @pl.when(j + 2 < ROWS_PER_SUB)
            def _():
                pltpu.async_copy(h_slab.at[sidx.at[j + 2]], rows_a, gs_a)

        _wait_scatter(rows_b, ss_b)
        plsc.subcore_barrier()

        pltpu.sync_copy(acc.at[pl.ds(sid * NODES_PER_S, NODES_PER_S)],
                        out_hbm.at[cid].at[pl.ds(sid * NODES_PER_S, NODES_PER_S)])

    return agg_kernel(h2, srcm, dstm)


def _split_halves(r):
    dh = r.shape[1] // 2
    return jnp.stack([r[:, :dh], r[:, dh:]])


def _mm_body(x_ref, w_ref, o_ref):
    r = jnp.dot(x_ref[...], w_ref[...], preferred_element_type=jnp.float32)
    o_ref[...] = _split_halves(r)


def _tc_matmul(x, w):
    return pl.pallas_call(
        _mm_body,
        out_shape=jax.ShapeDtypeStruct((2, x.shape[0], w.shape[1] // 2),
                                       jnp.float32),
    )(x, w)


def _mid_body(a_ref, b_ref, g_ref, be_ref, w_ref, o_ref):
    h = jnp.concatenate([a_ref[0, :N], a_ref[1, :N]], axis=1) + b_ref[...]
    mu = jnp.mean(h, axis=0, keepdims=True)
    var = jnp.mean((h - mu) ** 2, axis=0, keepdims=True)
    hn = (h - mu) / jnp.sqrt(var + 1e-5) * g_ref[...] + be_ref[...]
    hn = jnp.maximum(hn, 0.0)
    r = jnp.dot(hn, w_ref[...], preferred_element_type=jnp.float32)
    o_ref[...] = _split_halves(r)


def _tc_mid(a, b, g, be, w):
    return pl.pallas_call(
        _mid_body,
        out_shape=jax.ShapeDtypeStruct((2, N, w.shape[1] // 2), jnp.float32),
    )(a, b.reshape(1, -1), g.reshape(1, -1), be.reshape(1, -1), w)


def _fin_body(a_ref, b_ref, o_ref):
    h = jnp.concatenate([a_ref[0, :N], a_ref[1, :N]], axis=1) + b_ref[...]
    col = lax.broadcasted_iota(jnp.int32, h.shape, 1)
    valid = col < D_OUT
    m = jnp.max(jnp.where(valid, h, -jnp.inf), axis=1, keepdims=True)
    ex = jnp.where(valid, jnp.exp(h - m), 0.0)
    lse = m + jnp.log(jnp.sum(ex, axis=1, keepdims=True))
    o_ref[...] = h - lse


def _tc_final(a, b):
    return pl.pallas_call(
        _fin_body,
        out_shape=jax.ShapeDtypeStruct((N, D_PAD), jnp.float32),
    )(a, b.reshape(1, -1))


def kernel(x, edge_index, W0, b0, g0, be0, W1, b1, g1, be1, W2, b2):
    srcm = edge_index[0].reshape(NROWS, CH)
    dstm = edge_index[1].reshape(NROWS, CH)
    W2p = jnp.pad(W2, ((0, 0), (0, D_PAD - D_OUT)))
    b2p = jnp.pad(b2, (0, D_PAD - D_OUT))

    h0 = _tc_matmul(x, W0)
    a0 = _sc_aggregate(64, h0, srcm, dstm)
    h1 = _tc_mid(a0, b0, g0, be0, W1)
    a1 = _sc_aggregate(64, h1, srcm, dstm)
    h2 = _tc_mid(a1, b1, g1, be1, W2p)
    a2 = _sc_aggregate(32, h2, srcm, dstm)
    out = _tc_final(a2, b2p)
    return out[:, :D_OUT]


# speedup vs baseline: 1.5378x; 1.5378x over previous
import functools

import jax
import jax.numpy as jnp
from jax import lax
from jax.experimental import pallas as pl
from jax.experimental.pallas import tpu as pltpu
from jax.experimental.pallas import tpu_sc as plsc

N = 10000
E = 320000
D_IN = 128
D_HID = 128
D_OUT = 40
D_PAD = 64

CH = 125
NROWS = E // CH
ROWS_PER_SUB = NROWS // 16
NP = 10240
NODES_PER_S = NP // 16
ZCH = 128


def _sc_aggregate(dh, h2, srcm, dstm):
    mesh = plsc.VectorSubcoreMesh(core_axis_name="c", subcore_axis_name="s")

    @functools.partial(
        pl.kernel,
        out_type=jax.ShapeDtypeStruct((2, NP, dh), jnp.float32),
        mesh=mesh,
        compiler_params=pltpu.CompilerParams(use_tc_tiling_on_sc=False),
        scratch_types=[
            pltpu.VMEM((ROWS_PER_SUB, CH), jnp.int32),
            pltpu.VMEM((ROWS_PER_SUB, CH), jnp.int32),
            pltpu.VMEM((CH, dh), jnp.float32),
            pltpu.VMEM((CH, dh), jnp.float32),
            pltpu.VMEM((CH, dh), jnp.float32),
            pltpu.VMEM((CH, dh), jnp.float32),
            pltpu.VMEM((ZCH, dh), jnp.float32),
            pltpu.VMEM_SHARED((NP, dh), jnp.float32),
            pltpu.SemaphoreType.DMA,
            pltpu.SemaphoreType.DMA,
            pltpu.SemaphoreType.DMA,
            pltpu.SemaphoreType.DMA,
            pltpu.SemaphoreType.DMA,
            pltpu.SemaphoreType.DMA,
            pltpu.SemaphoreType.DMA,
            pltpu.SemaphoreType.DMA,
        ],
    )
    def agg_kernel(h_hbm, src_hbm, dst_hbm, out_hbm, sidx, didx,
                   r0, r1, r2, r3, zbuf, acc,
                   g0, g1, g2, g3, s0, s1, s2, s3):
        cid = lax.axis_index("c")
        sid = lax.axis_index("s")
        row0 = sid * ROWS_PER_SUB

        pltpu.sync_copy(src_hbm.at[pl.ds(row0, ROWS_PER_SUB)], sidx)
        pltpu.sync_copy(dst_hbm.at[pl.ds(row0, ROWS_PER_SUB)], didx)

        zv = jnp.zeros((16,), jnp.float32)

        @pl.loop(0, ZCH)
        def _(i):
            @pl.loop(0, dh, step=16)
            def _(j):
                zbuf[i, pl.ds(j, 16)] = zv

        @pl.loop(0, NODES_PER_S // ZCH)
        def _(k):
            pltpu.sync_copy(zbuf, acc.at[pl.ds(sid * NODES_PER_S + k * ZCH, ZCH)])

        plsc.subcore_barrier()

        h_slab = h_hbm.at[cid]
        bufs = (r0, r1, r2, r3)
        gsems = (g0, g1, g2, g3)
        ssems = (s0, s1, s2, s3)

        def _wait_g(buf, sem):
            pltpu.make_async_copy(h_slab.at[sidx.at[0]], buf, sem).wait()

        def _wait_s(buf, sem):
            pltpu.make_async_copy(buf, acc.at[didx.at[0]], sem).wait()

        for k in range(3):
            pltpu.async_copy(h_slab.at[sidx.at[k]], bufs[k], gsems[k])

        @pl.loop(0, ROWS_PER_SUB, step=4)
        def _(j):
            for k in range(4):
                i = j + k
                pk = (k + 3) % 4
                _wait_g(bufs[k], gsems[k])
                pltpu.async_copy(bufs[k], acc.at[didx.at[i]], ssems[k],
                                 add=True)
                if k == 0:
                    @pl.when(j > 0)
                    def _(pk=pk):
                        _wait_s(bufs[pk], ssems[pk])

                    pltpu.async_copy(h_slab.at[sidx.at[i + 3]], bufs[pk],
                                     gsems[pk])
                else:
                    _wait_s(bufs[pk], ssems[pk])

                    @pl.when(i + 3 < ROWS_PER_SUB)
                    def _(i=i, pk=pk):
                        pltpu.async_copy(h_slab.at[sidx.at[i + 3]], bufs[pk],
                                         gsems[pk])

        _wait_s(bufs[3], ssems[3])
        plsc.subcore_barrier()

        pltpu.sync_copy(acc.at[pl.ds(sid * NODES_PER_S, NODES_PER_S)],
                        out_hbm.at[cid].at[pl.ds(sid * NODES_PER_S, NODES_PER_S)])

    return agg_kernel(h2, srcm, dstm)


def _split_halves(r):
    dh = r.shape[1] // 2
    return jnp.stack([r[:, :dh], r[:, dh:]])


def _mm_body(x_ref, w_ref, o_ref):
    r = jnp.dot(x_ref[...], w_ref[...], preferred_element_type=jnp.float32)
    o_ref[...] = _split_halves(r)


def _tc_matmul(x, w):
    return pl.pallas_call(
        _mm_body,
        out_shape=jax.ShapeDtypeStruct((2, x.shape[0], w.shape[1] // 2),
                                       jnp.float32),
    )(x, w)


def _mid_body(a_ref, b_ref, g_ref, be_ref, w_ref, o_ref):
    h = jnp.concatenate([a_ref[0, :N], a_ref[1, :N]], axis=1) + b_ref[...]
    mu = jnp.mean(h, axis=0, keepdims=True)
    var = jnp.mean((h - mu) ** 2, axis=0, keepdims=True)
    hn = (h - mu) / jnp.sqrt(var + 1e-5) * g_ref[...] + be_ref[...]
    hn = jnp.maximum(hn, 0.0)
    r = jnp.dot(hn, w_ref[...], preferred_element_type=jnp.float32)
    o_ref[...] = _split_halves(r)


def _tc_mid(a, b, g, be, w):
    return pl.pallas_call(
        _mid_body,
        out_shape=jax.ShapeDtypeStruct((2, N, w.shape[1] // 2), jnp.float32),
    )(a, b.reshape(1, -1), g.reshape(1, -1), be.reshape(1, -1), w)


def _fin_body(a_ref, b_ref, o_ref):
    h = jnp.concatenate([a_ref[0, :N], a_ref[1, :N]], axis=1) + b_ref[...]
    col = lax.broadcasted_iota(jnp.int32, h.shape, 1)
    valid = col < D_OUT
    m = jnp.max(jnp.where(valid, h, -jnp.inf), axis=1, keepdims=True)
    ex = jnp.where(valid, jnp.exp(h - m), 0.0)
    lse = m + jnp.log(jnp.sum(ex, axis=1, keepdims=True))
    o_ref[...] = h - lse


def _tc_final(a, b):
    return pl.pallas_call(
        _fin_body,
        out_shape=jax.ShapeDtypeStruct((N, D_PAD), jnp.float32),
    )(a, b.reshape(1, -1))


def kernel(x, edge_index, W0, b0, g0, be0, W1, b1, g1, be1, W2, b2):
    srcm = edge_index[0].reshape(NROWS, CH)
    dstm = edge_index[1].reshape(NROWS, CH)
    W2p = jnp.pad(W2, ((0, 0), (0, D_PAD - D_OUT)))
    b2p = jnp.pad(b2, (0, D_PAD - D_OUT))

    h0 = _tc_matmul(x, W0)
    a0 = _sc_aggregate(64, h0, srcm, dstm)
    h1 = _tc_mid(a0, b0, g0, be0, W1)
    a1 = _sc_aggregate(64, h1, srcm, dstm)
    h2 = _tc_mid(a1, b1, g1, be1, W2p)
    a2 = _sc_aggregate(32, h2, srcm, dstm)
    out = _tc_final(a2, b2p)
    return out[:, :D_OUT]


# speedup vs baseline: 1.6745x; 1.0889x over previous
import functools

import jax
import jax.numpy as jnp
from jax import lax
from jax.experimental import pallas as pl
from jax.experimental.pallas import tpu as pltpu
from jax.experimental.pallas import tpu_sc as plsc

N = 10000
E = 320000
D_IN = 128
D_HID = 128
D_OUT = 40
D_PAD = 64

CH = 125
NROWS = E // CH
ROWS_PER_SUB = NROWS // 16
NP = 10240
NODES_PER_S = NP // 16
NBUF = 8
PFD = 6
IBLK = 32
NBLK = ROWS_PER_SUB // IBLK


def _sc_aggregate(dh, h2, srcm, dstm):
    mesh = plsc.VectorSubcoreMesh(core_axis_name="c", subcore_axis_name="s")

    @functools.partial(
        pl.kernel,
        out_type=jax.ShapeDtypeStruct((2, NP, dh), jnp.float32),
        mesh=mesh,
        compiler_params=pltpu.CompilerParams(use_tc_tiling_on_sc=False),
        scratch_types=[
            pltpu.VMEM((2, IBLK, CH), jnp.int32),
            pltpu.VMEM((2, IBLK, CH), jnp.int32),
            pltpu.VMEM_SHARED((NP, dh), jnp.float32),
        ]
        + [pltpu.VMEM((CH, dh), jnp.float32)] * NBUF
        + [pltpu.SemaphoreType.DMA] * (2 * NBUF + 2),
    )
    def agg_kernel(h_hbm, src_hbm, dst_hbm, out_hbm, sidx, didx, acc, *ring):
        bufs = ring[:NBUF]
        gsems = ring[NBUF:2 * NBUF]
        ssems = ring[2 * NBUF:3 * NBUF]
        isg, idg = ring[3 * NBUF], ring[3 * NBUF + 1]
        cid = lax.axis_index("c")
        sid = lax.axis_index("s")
        row0 = sid * ROWS_PER_SUB
        h_slab = h_hbm.at[cid]
        sslack = NBUF - PFD

        def _idx(ref, i):
            if isinstance(i, int):
                return ref.at[(i // IBLK) % 2, i % IBLK]
            blk = lax.div(i, IBLK)
            return ref.at[lax.rem(blk, 2), lax.rem(i, IBLK)]

        def _stage(blk, slot, sync=False):
            copies = [(src_hbm, sidx, isg), (dst_hbm, didx, idg)]
            for hbm, vref, sem in copies:
                cp = pltpu.async_copy(hbm.at[pl.ds(row0 + blk * IBLK, IBLK)],
                                      vref.at[slot], sem)
                if sync:
                    cp.wait()

        def _wait_stage():
            for vref, sem in ((sidx, isg), (didx, idg)):
                pltpu.make_async_copy(src_hbm.at[pl.ds(row0, IBLK)],
                                      vref.at[0], sem).wait()

        def _wait_g(buf, sem):
            pltpu.make_async_copy(h_slab.at[sidx.at[0, 0]], buf, sem).wait()

        def _wait_s(buf, sem):
            pltpu.make_async_copy(buf, acc.at[didx.at[0, 0]], sem).wait()

        _stage(0, 0, sync=True)
        _stage(1, 1)

        zv = jnp.zeros((16,), jnp.float32)
        zrows = NODES_PER_S // 8

        @pl.loop(0, zrows)
        def _(i):
            @pl.loop(0, dh, step=16)
            def _(j):
                bufs[NBUF - 1][i, pl.ds(j, 16)] = zv

        @pl.loop(0, 8)
        def _(k):
            pltpu.sync_copy(bufs[NBUF - 1].at[pl.ds(0, zrows)],
                            acc.at[pl.ds(sid * NODES_PER_S + k * zrows, zrows)])

        for k in range(PFD):
            pltpu.async_copy(h_slab.at[_idx(sidx, k)], bufs[k], gsems[k])

        plsc.subcore_barrier()

        @pl.loop(0, ROWS_PER_SUB, step=NBUF)
        def _(j):
            for k in range(NBUF):
                i = j + k
                _wait_g(bufs[k], gsems[k])
                pltpu.async_copy(bufs[k], acc.at[_idx(didx, i)], ssems[k],
                                 add=True)
                sk = (k - sslack) % NBUF
                if k >= sslack:
                    _wait_s(bufs[sk], ssems[sk])
                else:
                    @pl.when(j > 0)
                    def _(sk=sk):
                        _wait_s(bufs[sk], ssems[sk])

                gk = (k + PFD) % NBUF

                def _prefetch(i=i, gk=gk):
                    pltpu.async_copy(h_slab.at[_idx(sidx, i + PFD)],
                                     bufs[gk], gsems[gk])

                if k < NBUF - PFD:
                    _prefetch()
                else:
                    pl.when(i + PFD < ROWS_PER_SUB)(_prefetch)

                if k == 0:
                    @pl.when(jnp.logical_and(lax.rem(j, IBLK) == 8,
                                             jnp.logical_and(j >= IBLK + 8,
                                                             j < (NBLK - 1) * IBLK)))
                    def _():
                        nb = lax.div(j, IBLK) + 1
                        _stage(nb, lax.rem(nb, 2))

                    @pl.when(jnp.logical_and(lax.rem(j, IBLK) == IBLK - 8,
                                             j <= (NBLK - 1) * IBLK - 8))
                    def _():
                        _wait_stage()

        for t in range(sslack):
            b = (ROWS_PER_SUB - sslack + t) % NBUF
            _wait_s(bufs[b], ssems[b])
        plsc.subcore_barrier()

        pltpu.sync_copy(acc.at[pl.ds(sid * NODES_PER_S, NODES_PER_S)],
                        out_hbm.at[cid].at[pl.ds(sid * NODES_PER_S, NODES_PER_S)])

    return agg_kernel(h2, srcm, dstm)


def _split_halves(r):
    dh = r.shape[1] // 2
    return jnp.stack([r[:, :dh], r[:, dh:]])


def _mm_body(x_ref, w_ref, o_ref):
    r = jnp.dot(x_ref[...], w_ref[...], preferred_element_type=jnp.float32)
    o_ref[...] = _split_halves(r)


def _tc_matmul(x, w):
    return pl.pallas_call(
        _mm_body,
        out_shape=jax.ShapeDtypeStruct((2, x.shape[0], w.shape[1] // 2),
                                       jnp.float32),
    )(x, w)


def _mid_body(a_ref, b_ref, g_ref, be_ref, w_ref, o_ref):
    h = jnp.concatenate([a_ref[0, :N], a_ref[1, :N]], axis=1) + b_ref[...]
    mu = jnp.mean(h, axis=0, keepdims=True)
    var = jnp.mean((h - mu) ** 2, axis=0, keepdims=True)
    hn = (h - mu) / jnp.sqrt(var + 1e-5) * g_ref[...] + be_ref[...]
    hn = jnp.maximum(hn, 0.0)
    r = jnp.dot(hn, w_ref[...], preferred_element_type=jnp.float32)
    o_ref[...] = _split_halves(r)


def _tc_mid(a, b, g, be, w):
    return pl.pallas_call(
        _mid_body,
        out_shape=jax.ShapeDtypeStruct((2, N, w.shape[1] // 2), jnp.float32),
    )(a, b.reshape(1, -1), g.reshape(1, -1), be.reshape(1, -1), w)


def _fin_body(a_ref, b_ref, o_ref):
    h = jnp.concatenate([a_ref[0, :N], a_ref[1, :N]], axis=1) + b_ref[...]
    col = lax.broadcasted_iota(jnp.int32, h.shape, 1)
    valid = col < D_OUT
    m = jnp.max(jnp.where(valid, h, -jnp.inf), axis=1, keepdims=True)
    ex = jnp.where(valid, jnp.exp(h - m), 0.0)
    lse = m + jnp.log(jnp.sum(ex, axis=1, keepdims=True))
    o_ref[...] = h - lse


def _tc_final(a, b):
    return pl.pallas_call(
        _fin_body,
        out_shape=jax.ShapeDtypeStruct((N, D_PAD), jnp.float32),
    )(a, b.reshape(1, -1))


def kernel(x, edge_index, W0, b0, g0, be0, W1, b1, g1, be1, W2, b2):
    srcm = edge_index[0].reshape(NROWS, CH)
    dstm = edge_index[1].reshape(NROWS, CH)
    W2p = jnp.pad(W2, ((0, 0), (0, D_PAD - D_OUT)))
    b2p = jnp.pad(b2, (0, D_PAD - D_OUT))

    h0 = _tc_matmul(x, W0)
    a0 = _sc_aggregate(64, h0, srcm, dstm)
    h1 = _tc_mid(a0, b0, g0, be0, W1)
    a1 = _sc_aggregate(64, h1, srcm, dstm)
    h2 = _tc_mid(a1, b1, g1, be1, W2p)
    a2 = _sc_aggregate(32, h2, srcm, dstm)
    out = _tc_final(a2, b2p)
    return out[:, :D_OUT]
